# baseline (speedup 1.0000x reference)
import jax
import jax.numpy as jnp
from jax import lax
from jax.experimental import pallas as pl
from jax.experimental.pallas import tpu as pltpu
from jax.experimental.pallas import tpu_sc as plsc

NC, NS = 2, 16
NW = NC * NS
B = 16384
D = 64
BPW = B // NW
NIDX = BPW // 128


def _gather_cols(wt_hbm, idx_v, buf, sem):
    dmas = []
    for d in range(D):
        for j in range(NIDX):
            dmas.append(pltpu.async_copy(
                wt_hbm.at[d].at[idx_v.at[j]],
                buf.at[d, pl.ds(j * 128, 128)], sem))
    return dmas


def _sc_gmf_body(idx_hbm, wg_hbm, g_out, idx0_v, idx1_v, bufa, bufb,
                 sem_a, sem_b):
    wid = lax.axis_index("s") * NC + lax.axis_index("c")
    base = wid * BPW
    pltpu.sync_copy(idx_hbm.at[0, wid], idx0_v)
    pltpu.sync_copy(idx_hbm.at[1, wid], idx1_v)
    da = _gather_cols(wg_hbm, idx0_v, bufa, sem_a)
    db = _gather_cols(wg_hbm, idx1_v, bufb, sem_b)
    for x in da:
        x.wait()
    for x in db:
        x.wait()

    def prod_row(r, _):
        for c in range(BPW // 16):
            bufa[r, pl.ds(c * 16, 16)] = (
                bufa[r, pl.ds(c * 16, 16)] * bufb[r, pl.ds(c * 16, 16)])
        return _
    lax.fori_loop(0, D, prod_row, 0)
    pltpu.sync_copy(bufa, g_out.at[:, pl.ds(base, BPW)])


def _sc_mlp_body(idx_hbm, wm_hbm, m0_out, m1_out, idx0_v, idx1_v, bufa, bufb,
                 sem_a, sem_b):
    wid = lax.axis_index("s") * NC + lax.axis_index("c")
    base = wid * BPW
    pltpu.sync_copy(idx_hbm.at[0, wid], idx0_v)
    pltpu.sync_copy(idx_hbm.at[1, wid], idx1_v)
    da = _gather_cols(wm_hbm, idx0_v, bufa, sem_a)
    db = _gather_cols(wm_hbm, idx1_v, bufb, sem_b)
    for x in da:
        x.wait()
    pltpu.sync_copy(bufa, m0_out.at[:, pl.ds(base, BPW)])
    for x in db:
        x.wait()
    pltpu.sync_copy(bufb, m1_out.at[:, pl.ds(base, BPW)])


def _sc_call(body, n_out):
    mesh = plsc.VectorSubcoreMesh(core_axis_name="c", subcore_axis_name="s")
    return pl.kernel(
        body,
        out_type=tuple(
            jax.ShapeDtypeStruct((D, B), jnp.float32) for _ in range(n_out)),
        mesh=mesh,
        scratch_types=[
            pltpu.VMEM((NIDX, 128), jnp.int32),
            pltpu.VMEM((NIDX, 128), jnp.int32),
            pltpu.VMEM((D, BPW), jnp.float32),
            pltpu.VMEM((D, BPW), jnp.float32),
            pltpu.SemaphoreType.DMA,
            pltpu.SemaphoreType.DMA,
        ],
        compiler_params=pltpu.CompilerParams(use_tc_tiling_on_sc=False),
    )


@jax.jit
def _sc_gather(idx, wg_t, wm_t):
    g, = _sc_call(_sc_gmf_body, 1)(idx, wg_t)
    m0, m1 = _sc_call(_sc_mlp_body, 2)(idx, wm_t)
    return g, m0, m1


BLK = 2048


def _tc_body(g_ref, m0_ref, m1_ref, w0a, w0b, b0, w1t, b1, w2t, b2,
             wcg, wch, bc, out_ref):
    f32 = jnp.float32
    cdim0 = (((0,), (0,)), ((), ()))
    h = lax.dot_general(m0_ref[...], w0a[...], cdim0,
                        preferred_element_type=f32)
    h = h + lax.dot_general(m1_ref[...], w0b[...], cdim0,
                            preferred_element_type=f32)
    h = jnp.maximum(h + b0[...], 0.0)
    h = jnp.maximum(jnp.dot(h, w1t[...], preferred_element_type=f32) + b1[...], 0.0)
    h = jnp.maximum(jnp.dot(h, w2t[...], preferred_element_type=f32) + b2[...], 0.0)
    logit = (jnp.dot(h, wch[...], preferred_element_type=f32)
             + lax.dot_general(g_ref[...], wcg[...], cdim0,
                               preferred_element_type=f32)
             + bc[...])
    out_ref[...] = logit


@jax.jit
def _tc_mlp(g, m0, m1, w0a, w0b, b0, w1t, b1, w2t, b2, wcg, wch, bc):
    full = lambda shape: pl.BlockSpec(shape, lambda i: (0, 0))
    return pl.pallas_call(
        _tc_body,
        grid=(B // BLK,),
        in_specs=[
            pl.BlockSpec((D, BLK), lambda i: (0, i)),
            pl.BlockSpec((D, BLK), lambda i: (0, i)),
            pl.BlockSpec((D, BLK), lambda i: (0, i)),
            full(w0a.shape), full(w0b.shape), full(b0.shape),
            full(w1t.shape), full(b1.shape),
            full(w2t.shape), full(b2.shape),
            full(wcg.shape), full(wch.shape), full(bc.shape),
        ],
        out_specs=pl.BlockSpec((BLK, 1), lambda i: (i, 0)),
        out_shape=jax.ShapeDtypeStruct((B, 1), jnp.float32),
    )(g, m0, m1, w0a, w0b, b0, w1t, b1, w2t, b2, wcg, wch, bc)


@jax.jit
def _pipeline(sparse_features, W_gmf, W_mlp, Wl0, bl0, Wl1, bl1, Wl2, bl2, Wc, bc):
    idx = sparse_features.astype(jnp.int32).T.reshape(2, NW, NIDX, 128)
    g, m0, m1 = _sc_gather(idx, W_gmf.T, W_mlp.T)
    return _tc_mlp(
        g, m0, m1,
        Wl0[:, :D].T, Wl0[:, D:].T, bl0.reshape(1, -1),
        Wl1.T, bl1.reshape(1, -1),
        Wl2.T, bl2.reshape(1, -1),
        Wc[:, :D].T, Wc[:, D:].T, bc.reshape(1, 1),
    )


def kernel(sparse_features, W_gmf, W_mlp, Wl0, bl0, Wl1, bl1, Wl2, bl2, Wc, bc):
    return _pipeline(sparse_features, W_gmf, W_mlp, Wl0, bl0, Wl1, bl1, Wl2,
                     bl2, Wc, bc)


# speedup vs baseline: 14.4488x; 14.4488x over previous
import jax
import jax.numpy as jnp
from jax import lax
from jax.experimental import pallas as pl
from jax.experimental.pallas import tpu as pltpu
from jax.experimental.pallas import tpu_sc as plsc

NC, NS = 2, 16
NW = NC * NS
B = 16384
V = 1000000
D = 64
BPW = B // NW
NIDX = BPW // 128

PBLK = 2048


def _pack_body(wg_ref, wm_ref, eye_ref, out_ref):
    f32 = jnp.float32
    c00 = (((0,), (0,)), ((), ()))
    out_ref[:, 0:D] = lax.dot_general(wg_ref[...], eye_ref[...], c00,
                                      preferred_element_type=f32)
    out_ref[:, D:2 * D] = lax.dot_general(wm_ref[...], eye_ref[...], c00,
                                          preferred_element_type=f32)


@jax.jit
def _tc_pack(wg_t, wm_t, eye):
    return pl.pallas_call(
        _pack_body,
        grid=((V + PBLK - 1) // PBLK,),
        in_specs=[
            pl.BlockSpec((D, PBLK), lambda i: (0, i)),
            pl.BlockSpec((D, PBLK), lambda i: (0, i)),
            pl.BlockSpec((D, D), lambda i: (0, 0)),
        ],
        out_specs=pl.BlockSpec((PBLK, 2 * D), lambda i: (i, 0)),
        out_shape=jax.ShapeDtypeStruct((V, 2 * D), jnp.float32),
    )(wg_t, wm_t, eye)


def _sc_body(idx_hbm, pack_hbm, e0_out, e1_out, idx0_v, idx1_v, buf, sem):
    wid = lax.axis_index("s") * NC + lax.axis_index("c")
    base = wid * BPW
    pltpu.sync_copy(idx_hbm.at[0, wid], idx0_v)
    pltpu.sync_copy(idx_hbm.at[1, wid], idx1_v)
    dmas = []
    for j in range(NIDX):
        dmas.append(pltpu.async_copy(
            pack_hbm.at[idx0_v.at[j]], buf.at[pl.ds(j * 128, 128)], sem))
    for x in dmas:
        x.wait()
    pltpu.sync_copy(buf, e0_out.at[pl.ds(base, BPW)])
    dmas = []
    for j in range(NIDX):
        dmas.append(pltpu.async_copy(
            pack_hbm.at[idx1_v.at[j]], buf.at[pl.ds(j * 128, 128)], sem))
    for x in dmas:
        x.wait()
    pltpu.sync_copy(buf, e1_out.at[pl.ds(base, BPW)])


@jax.jit
def _sc_gather(idx, pack):
    mesh = plsc.VectorSubcoreMesh(core_axis_name="c", subcore_axis_name="s")
    f = pl.kernel(
        _sc_body,
        out_type=(
            jax.ShapeDtypeStruct((B, 2 * D), jnp.float32),
            jax.ShapeDtypeStruct((B, 2 * D), jnp.float32),
        ),
        mesh=mesh,
        scratch_types=[
            pltpu.VMEM((NIDX, 128), jnp.int32),
            pltpu.VMEM((NIDX, 128), jnp.int32),
            pltpu.VMEM((BPW, 2 * D), jnp.float32),
            pltpu.SemaphoreType.DMA,
        ],
        compiler_params=pltpu.CompilerParams(use_tc_tiling_on_sc=True),
    )
    return f(idx, pack)


BLK = 2048


def _tc_body(e0_ref, e1_ref, w0a, w0b, b0, w1t, b1, w2t, b2,
             wcg, wch, bc, out_ref):
    f32 = jnp.float32
    g = e0_ref[:, 0:D] * e1_ref[:, 0:D]
    h = jnp.dot(e0_ref[:, D:2 * D], w0a[...], preferred_element_type=f32)
    h = h + jnp.dot(e1_ref[:, D:2 * D], w0b[...], preferred_element_type=f32)
    h = jnp.maximum(h + b0[...], 0.0)
    h = jnp.maximum(jnp.dot(h, w1t[...], preferred_element_type=f32) + b1[...], 0.0)
    h = jnp.maximum(jnp.dot(h, w2t[...], preferred_element_type=f32) + b2[...], 0.0)
    logit = (jnp.dot(h, wch[...], preferred_element_type=f32)
             + jnp.dot(g, wcg[...], preferred_element_type=f32)
             + bc[...])
    out_ref[...] = logit


@jax.jit
def _tc_mlp(e0, e1, w0a, w0b, b0, w1t, b1, w2t, b2, wcg, wch, bc):
    full = lambda shape: pl.BlockSpec(shape, lambda i: (0, 0))
    return pl.pallas_call(
        _tc_body,
        grid=(B // BLK,),
        in_specs=[
            pl.BlockSpec((BLK, 2 * D), lambda i: (i, 0)),
            pl.BlockSpec((BLK, 2 * D), lambda i: (i, 0)),
            full(w0a.shape), full(w0b.shape), full(b0.shape),
            full(w1t.shape), full(b1.shape),
            full(w2t.shape), full(b2.shape),
            full(wcg.shape), full(wch.shape), full(bc.shape),
        ],
        out_specs=pl.BlockSpec((BLK, 1), lambda i: (i, 0)),
        out_shape=jax.ShapeDtypeStruct((B, 1), jnp.float32),
    )(e0, e1, w0a, w0b, b0, w1t, b1, w2t, b2, wcg, wch, bc)


@jax.jit
def _pipeline(sparse_features, W_gmf, W_mlp, Wl0, bl0, Wl1, bl1, Wl2, bl2, Wc, bc):
    idx = sparse_features.astype(jnp.int32).T.reshape(2, NW, NIDX, 128)
    pack = _tc_pack(W_gmf.T, W_mlp.T, jnp.eye(D, dtype=jnp.float32))
    e0, e1 = _sc_gather(idx, pack)
    return _tc_mlp(
        e0, e1,
        Wl0[:, :D].T, Wl0[:, D:].T, bl0.reshape(1, -1),
        Wl1.T, bl1.reshape(1, -1),
        Wl2.T, bl2.reshape(1, -1),
        Wc[:, :D].T, Wc[:, D:].T, bc.reshape(1, 1),
    )


def kernel(sparse_features, W_gmf, W_mlp, Wl0, bl0, Wl1, bl1, Wl2, bl2, Wc, bc):
    return _pipeline(sparse_features, W_gmf, W_mlp, Wl0, bl0, Wl1, bl1, Wl2,
                     bl2, Wc, bc)


# speedup vs baseline: 14.5075x; 1.0041x over previous
import jax
import jax.numpy as jnp
from jax import lax
from jax.experimental import pallas as pl
from jax.experimental.pallas import tpu as pltpu
from jax.experimental.pallas import tpu_sc as plsc

NC, NS = 2, 16
NW = NC * NS
B = 16384
V = 1000000
D = 64
BPW = B // NW
NIDX = BPW // 128

PBLK = 2048


def _pack_body(wg_ref, wm_ref, eye_ref, out_ref):
    del eye_ref
    out_ref[:, 0:D] = wg_ref[...].T
    out_ref[:, D:2 * D] = wm_ref[...].T


@jax.jit
def _tc_pack(wg_t, wm_t, eye):
    return pl.pallas_call(
        _pack_body,
        grid=((V + PBLK - 1) // PBLK,),
        in_specs=[
            pl.BlockSpec((D, PBLK), lambda i: (0, i)),
            pl.BlockSpec((D, PBLK), lambda i: (0, i)),
            pl.BlockSpec((D, D), lambda i: (0, 0)),
        ],
        out_specs=pl.BlockSpec((PBLK, 2 * D), lambda i: (i, 0)),
        out_shape=jax.ShapeDtypeStruct((V, 2 * D), jnp.float32),
    )(wg_t, wm_t, eye)


def _sc_body(idx_hbm, pack_hbm, e0_out, e1_out, idx0_v, idx1_v, buf, sem):
    wid = lax.axis_index("s") * NC + lax.axis_index("c")
    base = wid * BPW
    pltpu.sync_copy(idx_hbm.at[0, wid], idx0_v)
    pltpu.sync_copy(idx_hbm.at[1, wid], idx1_v)
    dmas = []
    for j in range(NIDX):
        dmas.append(pltpu.async_copy(
            pack_hbm.at[idx0_v.at[j]], buf.at[pl.ds(j * 128, 128)], sem))
    for x in dmas:
        x.wait()
    pltpu.sync_copy(buf, e0_out.at[pl.ds(base, BPW)])
    dmas = []
    for j in range(NIDX):
        dmas.append(pltpu.async_copy(
            pack_hbm.at[idx1_v.at[j]], buf.at[pl.ds(j * 128, 128)], sem))
    for x in dmas:
        x.wait()
    pltpu.sync_copy(buf, e1_out.at[pl.ds(base, BPW)])


@jax.jit
def _sc_gather(idx, pack):
    mesh = plsc.VectorSubcoreMesh(core_axis_name="c", subcore_axis_name="s")
    f = pl.kernel(
        _sc_body,
        out_type=(
            jax.ShapeDtypeStruct((B, 2 * D), jnp.float32),
            jax.ShapeDtypeStruct((B, 2 * D), jnp.float32),
        ),
        mesh=mesh,
        scratch_types=[
            pltpu.VMEM((NIDX, 128), jnp.int32),
            pltpu.VMEM((NIDX, 128), jnp.int32),
            pltpu.VMEM((BPW, 2 * D), jnp.float32),
            pltpu.SemaphoreType.DMA,
        ],
        compiler_params=pltpu.CompilerParams(use_tc_tiling_on_sc=True),
    )
    return f(idx, pack)


BLK = 2048


def _tc_body(e0_ref, e1_ref, w0a, w0b, b0, w1t, b1, w2t, b2,
             wcg, wch, bc, out_ref):
    f32 = jnp.float32
    g = e0_ref[:, 0:D] * e1_ref[:, 0:D]
    h = jnp.dot(e0_ref[:, D:2 * D], w0a[...], preferred_element_type=f32)
    h = h + jnp.dot(e1_ref[:, D:2 * D], w0b[...], preferred_element_type=f32)
    h = jnp.maximum(h + b0[...], 0.0)
    h = jnp.maximum(jnp.dot(h, w1t[...], preferred_element_type=f32) + b1[...], 0.0)
    h = jnp.maximum(jnp.dot(h, w2t[...], preferred_element_type=f32) + b2[...], 0.0)
    logit = (jnp.dot(h, wch[...], preferred_element_type=f32)
             + jnp.dot(g, wcg[...], preferred_element_type=f32)
             + bc[...])
    out_ref[...] = logit


@jax.jit
def _tc_mlp(e0, e1, w0a, w0b, b0, w1t, b1, w2t, b2, wcg, wch, bc):
    full = lambda shape: pl.BlockSpec(shape, lambda i: (0, 0))
    return pl.pallas_call(
        _tc_body,
        grid=(B // BLK,),
        in_specs=[
            pl.BlockSpec((BLK, 2 * D), lambda i: (i, 0)),
            pl.BlockSpec((BLK, 2 * D), lambda i: (i, 0)),
            full(w0a.shape), full(w0b.shape), full(b0.shape),
            full(w1t.shape), full(b1.shape),
            full(w2t.shape), full(b2.shape),
            full(wcg.shape), full(wch.shape), full(bc.shape),
        ],
        out_specs=pl.BlockSpec((BLK, 1), lambda i: (i, 0)),
        out_shape=jax.ShapeDtypeStruct((B, 1), jnp.float32),
    )(e0, e1, w0a, w0b, b0, w1t, b1, w2t, b2, wcg, wch, bc)


@jax.jit
def _pipeline(sparse_features, W_gmf, W_mlp, Wl0, bl0, Wl1, bl1, Wl2, bl2, Wc, bc):
    idx = sparse_features.astype(jnp.int32).T.reshape(2, NW, NIDX, 128)
    pack = _tc_pack(W_gmf.T, W_mlp.T, jnp.eye(D, dtype=jnp.float32))
    e0, e1 = _sc_gather(idx, pack)
    return _tc_mlp(
        e0, e1,
        Wl0[:, :D].T, Wl0[:, D:].T, bl0.reshape(1, -1),
        Wl1.T, bl1.reshape(1, -1),
        Wl2.T, bl2.reshape(1, -1),
        Wc[:, :D].T, Wc[:, D:].T, bc.reshape(1, 1),
    )


def kernel(sparse_features, W_gmf, W_mlp, Wl0, bl0, Wl1, bl1, Wl2, bl2, Wc, bc):
    return _pipeline(sparse_features, W_gmf, W_mlp, Wl0, bl0, Wl1, bl1, Wl2,
                     bl2, Wc, bc)


# speedup vs baseline: 20.4795x; 1.4116x over previous
import jax
import jax.numpy as jnp
from jax import lax
from jax.experimental import pallas as pl
from jax.experimental.pallas import tpu as pltpu
from jax.experimental.pallas import tpu_sc as plsc

NC, NS = 2, 16
NW = NC * NS
B = 16384
V = 1000000
D = 64
BPW = B // NW
NIDX = BPW // 128

PBLK = 8192


def _pack_body(wg_ref, wm_ref, eye_ref, out_ref):
    del eye_ref
    out_ref[:, 0:D] = wg_ref[...].T
    out_ref[:, D:2 * D] = wm_ref[...].T


@jax.jit
def _tc_pack(wg_t, wm_t, eye):
    return pl.pallas_call(
        _pack_body,
        grid=((V + PBLK - 1) // PBLK,),
        in_specs=[
            pl.BlockSpec((D, PBLK), lambda i: (0, i)),
            pl.BlockSpec((D, PBLK), lambda i: (0, i)),
            pl.BlockSpec((D, D), lambda i: (0, 0)),
        ],
        out_specs=pl.BlockSpec((PBLK, 2 * D), lambda i: (i, 0)),
        out_shape=jax.ShapeDtypeStruct((V, 2 * D), jnp.float32),
    )(wg_t, wm_t, eye)


def _sc_body(idx_hbm, pack_hbm, e0_out, e1_out, idx0_v, idx1_v, buf, sem):
    wid = lax.axis_index("s") * NC + lax.axis_index("c")
    base = wid * BPW
    pltpu.sync_copy(idx_hbm.at[0, wid], idx0_v)
    pltpu.sync_copy(idx_hbm.at[1, wid], idx1_v)
    dmas = []
    for j in range(NIDX):
        dmas.append(pltpu.async_copy(
            pack_hbm.at[idx0_v.at[j]], buf.at[pl.ds(j * 128, 128)], sem))
    for x in dmas:
        x.wait()
    pltpu.sync_copy(buf, e0_out.at[pl.ds(base, BPW)])
    dmas = []
    for j in range(NIDX):
        dmas.append(pltpu.async_copy(
            pack_hbm.at[idx1_v.at[j]], buf.at[pl.ds(j * 128, 128)], sem))
    for x in dmas:
        x.wait()
    pltpu.sync_copy(buf, e1_out.at[pl.ds(base, BPW)])


@jax.jit
def _sc_gather(idx, pack):
    mesh = plsc.VectorSubcoreMesh(core_axis_name="c", subcore_axis_name="s")
    f = pl.kernel(
        _sc_body,
        out_type=(
            jax.ShapeDtypeStruct((B, 2 * D), jnp.float32),
            jax.ShapeDtypeStruct((B, 2 * D), jnp.float32),
        ),
        mesh=mesh,
        scratch_types=[
            pltpu.VMEM((NIDX, 128), jnp.int32),
            pltpu.VMEM((NIDX, 128), jnp.int32),
            pltpu.VMEM((BPW, 2 * D), jnp.float32),
            pltpu.SemaphoreType.DMA,
        ],
        compiler_params=pltpu.CompilerParams(use_tc_tiling_on_sc=True),
    )
    return f(idx, pack)


BLK = 2048


def _tc_body(e0_ref, e1_ref, w0a, w0b, b0, w1t, b1, w2t, b2,
             wcg, wch, bc, out_ref):
    f32 = jnp.float32
    g = e0_ref[:, 0:D] * e1_ref[:, 0:D]
    h = jnp.dot(e0_ref[:, D:2 * D], w0a[...], preferred_element_type=f32)
    h = h + jnp.dot(e1_ref[:, D:2 * D], w0b[...], preferred_element_type=f32)
    h = jnp.maximum(h + b0[...], 0.0)
    h = jnp.maximum(jnp.dot(h, w1t[...], preferred_element_type=f32) + b1[...], 0.0)
    h = jnp.maximum(jnp.dot(h, w2t[...], preferred_element_type=f32) + b2[...], 0.0)
    logit = (jnp.dot(h, wch[...], preferred_element_type=f32)
             + jnp.dot(g, wcg[...], preferred_element_type=f32)
             + bc[...])
    out_ref[...] = logit


@jax.jit
def _tc_mlp(e0, e1, w0a, w0b, b0, w1t, b1, w2t, b2, wcg, wch, bc):
    full = lambda shape: pl.BlockSpec(shape, lambda i: (0, 0))
    return pl.pallas_call(
        _tc_body,
        grid=(B // BLK,),
        in_specs=[
            pl.BlockSpec((BLK, 2 * D), lambda i: (i, 0)),
            pl.BlockSpec((BLK, 2 * D), lambda i: (i, 0)),
            full(w0a.shape), full(w0b.shape), full(b0.shape),
            full(w1t.shape), full(b1.shape),
            full(w2t.shape), full(b2.shape),
            full(wcg.shape), full(wch.shape), full(bc.shape),
        ],
        out_specs=pl.BlockSpec((BLK, 1), lambda i: (i, 0)),
        out_shape=jax.ShapeDtypeStruct((B, 1), jnp.float32),
    )(e0, e1, w0a, w0b, b0, w1t, b1, w2t, b2, wcg, wch, bc)


@jax.jit
def _pipeline(sparse_features, W_gmf, W_mlp, Wl0, bl0, Wl1, bl1, Wl2, bl2, Wc, bc):
    idx = sparse_features.astype(jnp.int32).T.reshape(2, NW, NIDX, 128)
    pack = _tc_pack(W_gmf.T, W_mlp.T, jnp.eye(D, dtype=jnp.float32))
    e0, e1 = _sc_gather(idx, pack)
    return _tc_mlp(
        e0, e1,
        Wl0[:, :D].T, Wl0[:, D:].T, bl0.reshape(1, -1),
        Wl1.T, bl1.reshape(1, -1),
        Wl2.T, bl2.reshape(1, -1),
        Wc[:, :D].T, Wc[:, D:].T, bc.reshape(1, 1),
    )


def kernel(sparse_features, W_gmf, W_mlp, Wl0, bl0, Wl1, bl1, Wl2, bl2, Wc, bc):
    return _pipeline(sparse_features, W_gmf, W_mlp, Wl0, bl0, Wl1, bl1, Wl2,
                     bl2, Wc, bc)


# speedup vs baseline: 21.7820x; 1.0636x over previous
import jax
import jax.numpy as jnp
from jax import lax
from jax.experimental import pallas as pl
from jax.experimental.pallas import tpu as pltpu
from jax.experimental.pallas import tpu_sc as plsc

NC, NS = 2, 16
NW = NC * NS
B = 16384
V = 1000000
D = 64
BPW = B // NW
NIDX = BPW // 128

PBLK = 16384


def _pack_body(wg_ref, wm_ref, eye_ref, out_ref):
    del eye_ref
    out_ref[:, 0:D] = wg_ref[...].T
    out_ref[:, D:2 * D] = wm_ref[...].T


@jax.jit
def _tc_pack(wg_t, wm_t, eye):
    return pl.pallas_call(
        _pack_body,
        grid=((V + PBLK - 1) // PBLK,),
        in_specs=[
            pl.BlockSpec((D, PBLK), lambda i: (0, i)),
            pl.BlockSpec((D, PBLK), lambda i: (0, i)),
            pl.BlockSpec((D, D), lambda i: (0, 0)),
        ],
        out_specs=pl.BlockSpec((PBLK, 2 * D), lambda i: (i, 0)),
        out_shape=jax.ShapeDtypeStruct((V, 2 * D), jnp.float32),
    )(wg_t, wm_t, eye)


def _sc_body(idx_hbm, pack_hbm, e0_out, e1_out, idx0_v, idx1_v, buf, sem):
    wid = lax.axis_index("s") * NC + lax.axis_index("c")
    base = wid * BPW
    pltpu.sync_copy(idx_hbm.at[0, wid], idx0_v)
    pltpu.sync_copy(idx_hbm.at[1, wid], idx1_v)
    dmas = []
    for j in range(NIDX):
        dmas.append(pltpu.async_copy(
            pack_hbm.at[idx0_v.at[j]], buf.at[pl.ds(j * 128, 128)], sem))
    for x in dmas:
        x.wait()
    pltpu.sync_copy(buf, e0_out.at[pl.ds(base, BPW)])
    dmas = []
    for j in range(NIDX):
        dmas.append(pltpu.async_copy(
            pack_hbm.at[idx1_v.at[j]], buf.at[pl.ds(j * 128, 128)], sem))
    for x in dmas:
        x.wait()
    pltpu.sync_copy(buf, e1_out.at[pl.ds(base, BPW)])


@jax.jit
def _sc_gather(idx, pack):
    mesh = plsc.VectorSubcoreMesh(core_axis_name="c", subcore_axis_name="s")
    f = pl.kernel(
        _sc_body,
        out_type=(
            jax.ShapeDtypeStruct((B, 2 * D), jnp.float32),
            jax.ShapeDtypeStruct((B, 2 * D), jnp.float32),
        ),
        mesh=mesh,
        scratch_types=[
            pltpu.VMEM((NIDX, 128), jnp.int32),
            pltpu.VMEM((NIDX, 128), jnp.int32),
            pltpu.VMEM((BPW, 2 * D), jnp.float32),
            pltpu.SemaphoreType.DMA,
        ],
        compiler_params=pltpu.CompilerParams(use_tc_tiling_on_sc=True),
    )
    return f(idx, pack)


BLK = 2048


def _tc_body(e0_ref, e1_ref, w0a, w0b, b0, w1t, b1, w2t, b2,
             wcg, wch, bc, out_ref):
    f32 = jnp.float32
    g = e0_ref[:, 0:D] * e1_ref[:, 0:D]
    h = jnp.dot(e0_ref[:, D:2 * D], w0a[...], preferred_element_type=f32)
    h = h + jnp.dot(e1_ref[:, D:2 * D], w0b[...], preferred_element_type=f32)
    h = jnp.maximum(h + b0[...], 0.0)
    h = jnp.maximum(jnp.dot(h, w1t[...], preferred_element_type=f32) + b1[...], 0.0)
    h = jnp.maximum(jnp.dot(h, w2t[...], preferred_element_type=f32) + b2[...], 0.0)
    logit = (jnp.dot(h, wch[...], preferred_element_type=f32)
             + jnp.dot(g, wcg[...], preferred_element_type=f32)
             + bc[...])
    out_ref[...] = logit


@jax.jit
def _tc_mlp(e0, e1, w0a, w0b, b0, w1t, b1, w2t, b2, wcg, wch, bc):
    full = lambda shape: pl.BlockSpec(shape, lambda i: (0, 0))
    return pl.pallas_call(
        _tc_body,
        grid=(B // BLK,),
        in_specs=[
            pl.BlockSpec((BLK, 2 * D), lambda i: (i, 0)),
            pl.BlockSpec((BLK, 2 * D), lambda i: (i, 0)),
            full(w0a.shape), full(w0b.shape), full(b0.shape),
            full(w1t.shape), full(b1.shape),
            full(w2t.shape), full(b2.shape),
            full(wcg.shape), full(wch.shape), full(bc.shape),
        ],
        out_specs=pl.BlockSpec((BLK, 1), lambda i: (i, 0)),
        out_shape=jax.ShapeDtypeStruct((B, 1), jnp.float32),
    )(e0, e1, w0a, w0b, b0, w1t, b1, w2t, b2, wcg, wch, bc)


@jax.jit
def _pipeline(sparse_features, W_gmf, W_mlp, Wl0, bl0, Wl1, bl1, Wl2, bl2, Wc, bc):
    idx = sparse_features.astype(jnp.int32).T.reshape(2, NW, NIDX, 128)
    pack = _tc_pack(W_gmf.T, W_mlp.T, jnp.eye(D, dtype=jnp.float32))
    e0, e1 = _sc_gather(idx, pack)
    return _tc_mlp(
        e0, e1,
        Wl0[:, :D].T, Wl0[:, D:].T, bl0.reshape(1, -1),
        Wl1.T, bl1.reshape(1, -1),
        Wl2.T, bl2.reshape(1, -1),
        Wc[:, :D].T, Wc[:, D:].T, bc.reshape(1, 1),
    )


def kernel(sparse_features, W_gmf, W_mlp, Wl0, bl0, Wl1, bl1, Wl2, bl2, Wc, bc):
    return _pipeline(sparse_features, W_gmf, W_mlp, Wl0, bl0, Wl1, bl1, Wl2,
                     bl2, Wc, bc)


# speedup vs baseline: 21.9750x; 1.0089x over previous
import jax
import jax.numpy as jnp
from jax import lax
from jax.experimental import pallas as pl
from jax.experimental.pallas import tpu as pltpu
from jax.experimental.pallas import tpu_sc as plsc

NC, NS = 2, 16
NW = NC * NS
B = 16384
V = 1000000
D = 64
BPW = B // NW
NIDX = BPW // 128

PBLK = 20480


def _pack_body(wg_ref, wm_ref, eye_ref, out_ref):
    del eye_ref
    out_ref[:, 0:D] = wg_ref[...].T
    out_ref[:, D:2 * D] = wm_ref[...].T


@jax.jit
def _tc_pack(wg_t, wm_t, eye):
    return pl.pallas_call(
        _pack_body,
        grid=((V + PBLK - 1) // PBLK,),
        in_specs=[
            pl.BlockSpec((D, PBLK), lambda i: (0, i)),
            pl.BlockSpec((D, PBLK), lambda i: (0, i)),
            pl.BlockSpec((D, D), lambda i: (0, 0)),
        ],
        out_specs=pl.BlockSpec((PBLK, 2 * D), lambda i: (i, 0)),
        out_shape=jax.ShapeDtypeStruct((V, 2 * D), jnp.float32),
    )(wg_t, wm_t, eye)


def _sc_body(idx_hbm, pack_hbm, e0_out, e1_out, idx0_v, idx1_v, buf, sem):
    wid = lax.axis_index("s") * NC + lax.axis_index("c")
    base = wid * BPW
    pltpu.sync_copy(idx_hbm.at[0, wid], idx0_v)
    pltpu.sync_copy(idx_hbm.at[1, wid], idx1_v)
    dmas = []
    for j in range(NIDX):
        dmas.append(pltpu.async_copy(
            pack_hbm.at[idx0_v.at[j]], buf.at[pl.ds(j * 128, 128)], sem))
    for x in dmas:
        x.wait()
    pltpu.sync_copy(buf, e0_out.at[pl.ds(base, BPW)])
    dmas = []
    for j in range(NIDX):
        dmas.append(pltpu.async_copy(
            pack_hbm.at[idx1_v.at[j]], buf.at[pl.ds(j * 128, 128)], sem))
    for x in dmas:
        x.wait()
    pltpu.sync_copy(buf, e1_out.at[pl.ds(base, BPW)])


@jax.jit
def _sc_gather(idx, pack):
    mesh = plsc.VectorSubcoreMesh(core_axis_name="c", subcore_axis_name="s")
    f = pl.kernel(
        _sc_body,
        out_type=(
            jax.ShapeDtypeStruct((B, 2 * D), jnp.float32),
            jax.ShapeDtypeStruct((B, 2 * D), jnp.float32),
        ),
        mesh=mesh,
        scratch_types=[
            pltpu.VMEM((NIDX, 128), jnp.int32),
            pltpu.VMEM((NIDX, 128), jnp.int32),
            pltpu.VMEM((BPW, 2 * D), jnp.float32),
            pltpu.SemaphoreType.DMA,
        ],
        compiler_params=pltpu.CompilerParams(use_tc_tiling_on_sc=True),
    )
    return f(idx, pack)


BLK = 2048


def _tc_body(e0_ref, e1_ref, w0a, w0b, b0, w1t, b1, w2t, b2,
             wcg, wch, bc, out_ref):
    f32 = jnp.float32
    g = e0_ref[:, 0:D] * e1_ref[:, 0:D]
    h = jnp.dot(e0_ref[:, D:2 * D], w0a[...], preferred_element_type=f32)
    h = h + jnp.dot(e1_ref[:, D:2 * D], w0b[...], preferred_element_type=f32)
    h = jnp.maximum(h + b0[...], 0.0)
    h = jnp.maximum(jnp.dot(h, w1t[...], preferred_element_type=f32) + b1[...], 0.0)
    h = jnp.maximum(jnp.dot(h, w2t[...], preferred_element_type=f32) + b2[...], 0.0)
    logit = (jnp.dot(h, wch[...], preferred_element_type=f32)
             + jnp.dot(g, wcg[...], preferred_element_type=f32)
             + bc[...])
    out_ref[...] = logit


@jax.jit
def _tc_mlp(e0, e1, w0a, w0b, b0, w1t, b1, w2t, b2, wcg, wch, bc):
    full = lambda shape: pl.BlockSpec(shape, lambda i: (0, 0))
    return pl.pallas_call(
        _tc_body,
        grid=(B // BLK,),
        in_specs=[
            pl.BlockSpec((BLK, 2 * D), lambda i: (i, 0)),
            pl.BlockSpec((BLK, 2 * D), lambda i: (i, 0)),
            full(w0a.shape), full(w0b.shape), full(b0.shape),
            full(w1t.shape), full(b1.shape),
            full(w2t.shape), full(b2.shape),
            full(wcg.shape), full(wch.shape), full(bc.shape),
        ],
        out_specs=pl.BlockSpec((BLK, 1), lambda i: (i, 0)),
        out_shape=jax.ShapeDtypeStruct((B, 1), jnp.float32),
    )(e0, e1, w0a, w0b, b0, w1t, b1, w2t, b2, wcg, wch, bc)


@jax.jit
def _pipeline(sparse_features, W_gmf, W_mlp, Wl0, bl0, Wl1, bl1, Wl2, bl2, Wc, bc):
    idx = sparse_features.astype(jnp.int32).T.reshape(2, NW, NIDX, 128)
    pack = _tc_pack(W_gmf.T, W_mlp.T, jnp.eye(D, dtype=jnp.float32))
    e0, e1 = _sc_gather(idx, pack)
    return _tc_mlp(
        e0, e1,
        Wl0[:, :D].T, Wl0[:, D:].T, bl0.reshape(1, -1),
        Wl1.T, bl1.reshape(1, -1),
        Wl2.T, bl2.reshape(1, -1),
        Wc[:, :D].T, Wc[:, D:].T, bc.reshape(1, 1),
    )


def kernel(sparse_features, W_gmf, W_mlp, Wl0, bl0, Wl1, bl1, Wl2, bl2, Wc, bc):
    return _pipeline(sparse_features, W_gmf, W_mlp, Wl0, bl0, Wl1, bl1, Wl2,
                     bl2, Wc, bc)


# speedup vs baseline: 21.9926x; 1.0008x over previous
import jax
import jax.numpy as jnp
from jax import lax
from jax.experimental import pallas as pl
from jax.experimental.pallas import tpu as pltpu
from jax.experimental.pallas import tpu_sc as plsc

NC, NS = 2, 16
NW = NC * NS
B = 16384
V = 1000000
D = 64
BPW = B // NW
NIDX = BPW // 128

PBLK = 20480


def _pack_body(wg_ref, wm_ref, eye_ref, out_ref):
    del eye_ref
    out_ref[:, 0:D] = wg_ref[...].T
    out_ref[:, D:2 * D] = wm_ref[...].T


@jax.jit
def _tc_pack(wg_t, wm_t, eye):
    return pl.pallas_call(
        _pack_body,
        grid=((V + PBLK - 1) // PBLK,),
        in_specs=[
            pl.BlockSpec((D, PBLK), lambda i: (0, i)),
            pl.BlockSpec((D, PBLK), lambda i: (0, i)),
            pl.BlockSpec((D, D), lambda i: (0, 0)),
        ],
        out_specs=pl.BlockSpec((PBLK, 2 * D), lambda i: (i, 0)),
        out_shape=jax.ShapeDtypeStruct((V, 2 * D), jnp.float32),
    )(wg_t, wm_t, eye)


def _sc_body(idx_hbm, pack_hbm, e0_out, e1_out, idx0_v, idx1_v, buf, sem):
    wid = lax.axis_index("s") * NC + lax.axis_index("c")
    base = wid * BPW
    pltpu.sync_copy(idx_hbm.at[0, wid], idx0_v)
    pltpu.sync_copy(idx_hbm.at[1, wid], idx1_v)
    dmas = []
    for j in range(NIDX):
        dmas.append(pltpu.async_copy(
            pack_hbm.at[idx0_v.at[j]], buf.at[pl.ds(j * 128, 128)], sem))
    for x in dmas:
        x.wait()
    pltpu.sync_copy(buf, e0_out.at[pl.ds(base, BPW)])
    dmas = []
    for j in range(NIDX):
        dmas.append(pltpu.async_copy(
            pack_hbm.at[idx1_v.at[j]], buf.at[pl.ds(j * 128, 128)], sem))
    for x in dmas:
        x.wait()
    pltpu.sync_copy(buf, e1_out.at[pl.ds(base, BPW)])


@jax.jit
def _sc_gather(idx, pack):
    mesh = plsc.VectorSubcoreMesh(core_axis_name="c", subcore_axis_name="s")
    f = pl.kernel(
        _sc_body,
        out_type=(
            jax.ShapeDtypeStruct((B, 2 * D), jnp.float32),
            jax.ShapeDtypeStruct((B, 2 * D), jnp.float32),
        ),
        mesh=mesh,
        scratch_types=[
            pltpu.VMEM((NIDX, 128), jnp.int32),
            pltpu.VMEM((NIDX, 128), jnp.int32),
            pltpu.VMEM((BPW, 2 * D), jnp.float32),
            pltpu.SemaphoreType.DMA,
        ],
        compiler_params=pltpu.CompilerParams(use_tc_tiling_on_sc=True),
    )
    return f(idx, pack)


BLK = 8192


def _tc_body(e0_ref, e1_ref, w0a, w0b, b0, w1t, b1, w2t, b2,
             wcg, wch, bc, out_ref):
    f32 = jnp.float32
    g = e0_ref[:, 0:D] * e1_ref[:, 0:D]
    h = jnp.dot(e0_ref[:, D:2 * D], w0a[...], preferred_element_type=f32)
    h = h + jnp.dot(e1_ref[:, D:2 * D], w0b[...], preferred_element_type=f32)
    h = jnp.maximum(h + b0[...], 0.0)
    h = jnp.maximum(jnp.dot(h, w1t[...], preferred_element_type=f32) + b1[...], 0.0)
    h = jnp.maximum(jnp.dot(h, w2t[...], preferred_element_type=f32) + b2[...], 0.0)
    logit = (jnp.dot(h, wch[...], preferred_element_type=f32)
             + jnp.dot(g, wcg[...], preferred_element_type=f32)
             + bc[...])
    out_ref[...] = logit


@jax.jit
def _tc_mlp(e0, e1, w0a, w0b, b0, w1t, b1, w2t, b2, wcg, wch, bc):
    full = lambda shape: pl.BlockSpec(shape, lambda i: (0, 0))
    return pl.pallas_call(
        _tc_body,
        grid=(B // BLK,),
        in_specs=[
            pl.BlockSpec((BLK, 2 * D), lambda i: (i, 0)),
            pl.BlockSpec((BLK, 2 * D), lambda i: (i, 0)),
            full(w0a.shape), full(w0b.shape), full(b0.shape),
            full(w1t.shape), full(b1.shape),
            full(w2t.shape), full(b2.shape),
            full(wcg.shape), full(wch.shape), full(bc.shape),
        ],
        out_specs=pl.BlockSpec((BLK, 1), lambda i: (i, 0)),
        out_shape=jax.ShapeDtypeStruct((B, 1), jnp.float32),
    )(e0, e1, w0a, w0b, b0, w1t, b1, w2t, b2, wcg, wch, bc)


@jax.jit
def _pipeline(sparse_features, W_gmf, W_mlp, Wl0, bl0, Wl1, bl1, Wl2, bl2, Wc, bc):
    idx = sparse_features.astype(jnp.int32).T.reshape(2, NW, NIDX, 128)
    pack = _tc_pack(W_gmf.T, W_mlp.T, jnp.eye(D, dtype=jnp.float32))
    e0, e1 = _sc_gather(idx, pack)
    return _tc_mlp(
        e0, e1,
        Wl0[:, :D].T, Wl0[:, D:].T, bl0.reshape(1, -1),
        Wl1.T, bl1.reshape(1, -1),
        Wl2.T, bl2.reshape(1, -1),
        Wc[:, :D].T, Wc[:, D:].T, bc.reshape(1, 1),
    )


def kernel(sparse_features, W_gmf, W_mlp, Wl0, bl0, Wl1, bl1, Wl2, bl2, Wc, bc):
    return _pipeline(sparse_features, W_gmf, W_mlp, Wl0, bl0, Wl1, bl1, Wl2,
                     bl2, Wc, bc)


# speedup vs baseline: 22.0215x; 1.0013x over previous
import jax
import jax.numpy as jnp
from jax import lax
from jax.experimental import pallas as pl
from jax.experimental.pallas import tpu as pltpu
from jax.experimental.pallas import tpu_sc as plsc

NC, NS = 2, 16
NW = NC * NS
B = 16384
V = 1000000
D = 64
BPW = B // NW
NIDX = BPW // 128

PBLK = 20480


def _pack_body(wg_ref, wm_ref, out_ref):
    out_ref[:, 0:D] = wg_ref[...].T
    out_ref[:, D:2 * D] = wm_ref[...].T


@jax.jit
def _tc_pack(wg_t, wm_t):
    return pl.pallas_call(
        _pack_body,
        grid=((V + PBLK - 1) // PBLK,),
        in_specs=[
            pl.BlockSpec((D, PBLK), lambda i: (0, i)),
            pl.BlockSpec((D, PBLK), lambda i: (0, i)),
        ],
        out_specs=pl.BlockSpec((PBLK, 2 * D), lambda i: (i, 0)),
        out_shape=jax.ShapeDtypeStruct((V, 2 * D), jnp.float32),
    )(wg_t, wm_t)


def _sc_body(idx_hbm, pack_hbm, e0_out, e1_out, idx0_v, idx1_v, buf, sem):
    wid = lax.axis_index("s") * NC + lax.axis_index("c")
    base = wid * BPW
    pltpu.sync_copy(idx_hbm.at[0, wid], idx0_v)
    pltpu.sync_copy(idx_hbm.at[1, wid], idx1_v)
    dmas = []
    for j in range(NIDX):
        dmas.append(pltpu.async_copy(
            pack_hbm.at[idx0_v.at[j]], buf.at[pl.ds(j * 128, 128)], sem))
    for x in dmas:
        x.wait()
    pltpu.sync_copy(buf, e0_out.at[pl.ds(base, BPW)])
    dmas = []
    for j in range(NIDX):
        dmas.append(pltpu.async_copy(
            pack_hbm.at[idx1_v.at[j]], buf.at[pl.ds(j * 128, 128)], sem))
    for x in dmas:
        x.wait()
    pltpu.sync_copy(buf, e1_out.at[pl.ds(base, BPW)])


@jax.jit
def _sc_gather(idx, pack):
    mesh = plsc.VectorSubcoreMesh(core_axis_name="c", subcore_axis_name="s")
    f = pl.kernel(
        _sc_body,
        out_type=(
            jax.ShapeDtypeStruct((B, 2 * D), jnp.float32),
            jax.ShapeDtypeStruct((B, 2 * D), jnp.float32),
        ),
        mesh=mesh,
        scratch_types=[
            pltpu.VMEM((NIDX, 128), jnp.int32),
            pltpu.VMEM((NIDX, 128), jnp.int32),
            pltpu.VMEM((BPW, 2 * D), jnp.float32),
            pltpu.SemaphoreType.DMA,
        ],
        compiler_params=pltpu.CompilerParams(use_tc_tiling_on_sc=True),
    )
    return f(idx, pack)


BLK = 2048


def _tc_body(e0_ref, e1_ref, w0a, w0b, b0, w1t, b1, w2t, b2,
             wcg, wch, bc, out_ref):
    f32 = jnp.float32
    g = e0_ref[:, 0:D] * e1_ref[:, 0:D]
    h = jnp.dot(e0_ref[:, D:2 * D], w0a[...], preferred_element_type=f32)
    h = h + jnp.dot(e1_ref[:, D:2 * D], w0b[...], preferred_element_type=f32)
    h = jnp.maximum(h + b0[...], 0.0)
    h = jnp.maximum(jnp.dot(h, w1t[...], preferred_element_type=f32) + b1[...], 0.0)
    h = jnp.maximum(jnp.dot(h, w2t[...], preferred_element_type=f32) + b2[...], 0.0)
    logit = (jnp.dot(h, wch[...], preferred_element_type=f32)
             + jnp.dot(g, wcg[...], preferred_element_type=f32)
             + bc[...])
    out_ref[...] = logit


@jax.jit
def _tc_mlp(e0, e1, w0a, w0b, b0, w1t, b1, w2t, b2, wcg, wch, bc):
    full = lambda shape: pl.BlockSpec(shape, lambda i: (0, 0))
    return pl.pallas_call(
        _tc_body,
        grid=(B // BLK,),
        in_specs=[
            pl.BlockSpec((BLK, 2 * D), lambda i: (i, 0)),
            pl.BlockSpec((BLK, 2 * D), lambda i: (i, 0)),
            full(w0a.shape), full(w0b.shape), full(b0.shape),
            full(w1t.shape), full(b1.shape),
            full(w2t.shape), full(b2.shape),
            full(wcg.shape), full(wch.shape), full(bc.shape),
        ],
        out_specs=pl.BlockSpec((BLK, 1), lambda i: (i, 0)),
        out_shape=jax.ShapeDtypeStruct((B, 1), jnp.float32),
    )(e0, e1, w0a, w0b, b0, w1t, b1, w2t, b2, wcg, wch, bc)


@jax.jit
def _pipeline(sparse_features, W_gmf, W_mlp, Wl0, bl0, Wl1, bl1, Wl2, bl2, Wc, bc):
    idx = sparse_features.astype(jnp.int32).T.reshape(2, NW, NIDX, 128)
    pack = _tc_pack(W_gmf.T, W_mlp.T)
    e0, e1 = _sc_gather(idx, pack)
    return _tc_mlp(
        e0, e1,
        Wl0[:, :D].T, Wl0[:, D:].T, bl0.reshape(1, -1),
        Wl1.T, bl1.reshape(1, -1),
        Wl2.T, bl2.reshape(1, -1),
        Wc[:, :D].T, Wc[:, D:].T, bc.reshape(1, 1),
    )


def kernel(sparse_features, W_gmf, W_mlp, Wl0, bl0, Wl1, bl1, Wl2, bl2, Wc, bc):
    return _pipeline(sparse_features, W_gmf, W_mlp, Wl0, bl0, Wl1, bl1, Wl2,
                     bl2, Wc, bc)
